# baseline (speedup 1.0000x reference)
import functools

import jax
import jax.numpy as jnp
from jax import lax
from jax.experimental import pallas as pl
from jax.experimental.pallas import tpu as pltpu
from jax.experimental.pallas import tpu_sc as plsc

VOCAB = 1000000
IN_SIZE = 64
OUT_SIZE = 2
SEQ_LEN = 200
BATCH = 16384


_VBLK = 65536


def _round_bf16_bits(x):
    bits = lax.bitcast_convert_type(x, jnp.int32)
    return bits + 0x7FFF + lax.bitwise_and(
        lax.shift_right_logical(bits, 16), 1
    )


def _tw_body(w_ref, tblt_ref, out_ref):
    res = lax.dot_general(
        w_ref[...],
        tblt_ref[...],
        dimension_numbers=(((1,), (0,)), ((), ())),
        preferred_element_type=jnp.float32,
    )
    hi = lax.bitwise_and(_round_bf16_bits(res[0, :]), -65536)
    lo = lax.shift_right_logical(_round_bf16_bits(res[1, :]), 16)
    out_ref[...] = lax.bitwise_or(hi, lo)


def _compute_tw(w_scaled, table_t):
    return pl.pallas_call(
        _tw_body,
        grid=(pl.cdiv(VOCAB, _VBLK),),
        in_specs=[
            pl.BlockSpec((OUT_SIZE, IN_SIZE), lambda i: (0, 0)),
            pl.BlockSpec((IN_SIZE, _VBLK), lambda i: (0, i)),
        ],
        out_specs=pl.BlockSpec((_VBLK,), lambda i: (i,)),
        out_shape=jax.ShapeDtypeStruct((VOCAB,), jnp.int32),
    )(w_scaled, table_t)


_NC = 2
_NS = 16
_NW = _NC * _NS
_BPW = BATCH // _NW
_L = 16
_GC = 4
_GE = _GC * _BPW
_NCHUNK = SEQ_LEN // _GC
_NRING = 8


_TWSEG = 62496
_TWRND = 8928


def _sc_body(tw_hbm, xb_hbm, bias_hbm, out0_hbm, out1_hbm,
             idx_v, rows, accs, bias_v, tw_sp, stage_v, fill_sems, sems):
    sid = lax.axis_index("s")
    wid = sid * _NC + lax.axis_index("c")

    pltpu.sync_copy(bias_hbm, bias_v)

    def stage_round(base, n):
        pltpu.sync_copy(tw_hbm.at[pl.ds(base, n)], stage_v.at[pl.ds(0, n)])
        pltpu.sync_copy(stage_v.at[pl.ds(0, n)], tw_sp.at[pl.ds(base, n)])

    def stage(r, carry):
        stage_round(sid * _TWSEG + r * _TWRND, _TWRND)
        return carry

    lax.fori_loop(0, _TWSEG // _TWRND, stage, 0)

    @pl.when(sid == _NS - 1)
    def _():
        stage_round(_NS * _TWSEG, VOCAB - _NS * _TWSEG)

    def fill(c, slot):
        ring = lax.rem(c, _NRING) * _GE
        for j in range(_GC):
            l = c * _GC + j
            pltpu.async_copy(
                xb_hbm.at[l, wid, :],
                idx_v.at[pl.ds(ring + j * _BPW, _BPW)],
                fill_sems[slot],
            )

    def fill_wait(slot):
        for j in range(_GC):
            pltpu.make_async_copy(
                xb_hbm.at[0, 0, :], idx_v.at[pl.ds(0, _BPW)], fill_sems[slot]
            ).wait()

    fill(0, 0)
    fill(1, 1)

    for p in range(2):
        bvec = bias_v[p, :]
        for i in range(_BPW // _L):
            accs[p][pl.ds(i * _L, _L)] = bvec

    def gather(c, slot):
        ring = lax.rem(c, _NRING) * _GE
        pltpu.async_copy(
            tw_sp.at[idx_v.at[pl.ds(ring, _GE)]], rows[slot], sems[slot]
        )

    def wait(slot):
        pltpu.make_async_copy(
            tw_sp.at[idx_v.at[pl.ds(0, _GE)]], rows[slot], sems[slot]
        ).wait()

    def accum(slot):
        for g in range(_GC):
            for i in range(_BPW // _L):
                v = rows[slot][pl.ds(g * _BPW + i * _L, _L)]
                hi = plsc.bitcast(lax.bitwise_and(v, -65536), jnp.float32)
                lo = plsc.bitcast(lax.shift_left(v, 16), jnp.float32)
                plsc.addupdate(accs[0].at[pl.ds(i * _L, _L)], hi)
                plsc.addupdate(accs[1].at[pl.ds(i * _L, _L)], lo)

    plsc.subcore_barrier()

    fill_wait(0)
    gather(0, 0)
    fill(2, 0)
    fill_wait(1)
    gather(1, 1)
    fill(3, 1)

    def body(i, carry):
        c0 = i * 2
        for slot in range(2):
            c = c0 + slot
            wait(slot)
            accum(slot)

            @pl.when(c + 2 < _NCHUNK)
            def _():
                fill_wait(slot)
                gather(c + 2, slot)

            @pl.when(c + 4 < _NCHUNK)
            def _():
                fill(c + 4, slot)

        return carry

    lax.fori_loop(0, _NCHUNK // 2, body, 0)

    pltpu.sync_copy(accs[0], out0_hbm.at[pl.ds(wid * _BPW, _BPW)])
    pltpu.sync_copy(accs[1], out1_hbm.at[pl.ds(wid * _BPW, _BPW)])


@functools.partial(
    pl.kernel,
    out_type=[
        jax.ShapeDtypeStruct((BATCH,), jnp.float32),
        jax.ShapeDtypeStruct((BATCH,), jnp.float32),
    ],
    mesh=plsc.VectorSubcoreMesh(core_axis_name="c", subcore_axis_name="s"),
    compiler_params=pltpu.CompilerParams(needs_layout_passes=False),
    scratch_types=[
        pltpu.VMEM((_NRING * _GE,), jnp.int32),
        pltpu.VMEM((_GE,), jnp.int32),
        pltpu.VMEM((_GE,), jnp.int32),
        pltpu.VMEM((_BPW,), jnp.float32),
        pltpu.VMEM((_BPW,), jnp.float32),
        pltpu.VMEM((2, _L), jnp.float32),
        pltpu.VMEM_SHARED((VOCAB,), jnp.int32),
        pltpu.VMEM((_TWRND,), jnp.int32),
        pltpu.SemaphoreType.DMA,
        pltpu.SemaphoreType.DMA,
        pltpu.SemaphoreType.DMA,
        pltpu.SemaphoreType.DMA,
    ],
)
def _sc_gather(tw, xb3, bias, out0, out1, idx_v, r0, r1, acc0, acc1,
               bias_v, tw_sp, stage_v, fsem0, fsem1, sem0, sem1):
    _sc_body(tw, xb3, bias, out0, out1, idx_v, (r0, r1), (acc0, acc1),
             bias_v, tw_sp, stage_v, (fsem0, fsem1), (sem0, sem1))


@jax.jit
def _run(xb, table, W, b):
    w_scaled = W * (1.0 / SEQ_LEN)
    tw = _compute_tw(w_scaled, table.T)
    xb3 = xb.reshape(SEQ_LEN, _NW, _BPW)
    bias2 = jnp.broadcast_to(b[:, None], (OUT_SIZE, _L))
    out0, out1 = _sc_gather(tw, xb3, bias2)
    return jnp.stack([out0, out1], axis=-1)


def kernel(xb, table, W, b):
    return _run(xb, table, W, b)


# speedup vs baseline: 1.0412x; 1.0412x over previous
import functools

import jax
import jax.numpy as jnp
from jax import lax
from jax.experimental import pallas as pl
from jax.experimental.pallas import tpu as pltpu
from jax.experimental.pallas import tpu_sc as plsc

VOCAB = 1000000
IN_SIZE = 64
OUT_SIZE = 2
SEQ_LEN = 200
BATCH = 16384


_VBLK = 32768


def _round_bf16_bits(x):
    bits = lax.bitcast_convert_type(x, jnp.int32)
    return bits + 0x7FFF + lax.bitwise_and(
        lax.shift_right_logical(bits, 16), 1
    )


def _tw_body(w_ref, tblt_ref, out_ref):
    res = lax.dot_general(
        w_ref[...],
        tblt_ref[...],
        dimension_numbers=(((1,), (0,)), ((), ())),
        preferred_element_type=jnp.float32,
    )
    hi = lax.bitwise_and(_round_bf16_bits(res[0, :]), -65536)
    lo = lax.shift_right_logical(_round_bf16_bits(res[1, :]), 16)
    out_ref[...] = lax.bitwise_or(hi, lo)


def _compute_tw(w_scaled, table_t):
    return pl.pallas_call(
        _tw_body,
        grid=(pl.cdiv(VOCAB, _VBLK),),
        in_specs=[
            pl.BlockSpec((OUT_SIZE, IN_SIZE), lambda i: (0, 0)),
            pl.BlockSpec((IN_SIZE, _VBLK), lambda i: (0, i)),
        ],
        out_specs=pl.BlockSpec((_VBLK,), lambda i: (i,)),
        out_shape=jax.ShapeDtypeStruct((VOCAB,), jnp.int32),
    )(w_scaled, table_t)


_NC = 2
_NS = 16
_NW = _NC * _NS
_BPW = BATCH // _NW
_L = 16
_GC = 4
_GE = _GC * _BPW
_NCHUNK = SEQ_LEN // _GC
_NRING = 8


_TWSEG = 62496
_TWRND = 8928


def _sc_body(tw_hbm, xb_hbm, bias_hbm, out0_hbm, out1_hbm,
             idx_v, rows, accs, bias_v, tw_sp, stage_v, fill_sems, sems,
             stage_sems):
    sid = lax.axis_index("s")
    wid = sid * _NC + lax.axis_index("c")


    def fill(c, slot):
        ring = lax.rem(c, _NRING) * _GE
        for j in range(_GC):
            l = c * _GC + j
            pltpu.async_copy(
                xb_hbm.at[l, wid, :],
                idx_v.at[pl.ds(ring + j * _BPW, _BPW)],
                fill_sems[slot],
            )

    def fill_wait(slot):
        for j in range(_GC):
            pltpu.make_async_copy(
                xb_hbm.at[0, 0, :], idx_v.at[pl.ds(0, _BPW)], fill_sems[slot]
            ).wait()

    fill(0, 0)
    fill(1, 1)
    fill(2, 0)
    fill(3, 1)

    nst = _TWSEG // _TWRND
    sbufs = (stage_v.at[pl.ds(0, _TWRND)], stage_v.at[pl.ds(_TWRND, _TWRND)])

    def sin(r, buf):
        pltpu.async_copy(
            tw_hbm.at[pl.ds(sid * _TWSEG + r * _TWRND, _TWRND)],
            sbufs[buf],
            stage_sems[buf],
        )

    def sin_wait(buf):
        pltpu.make_async_copy(
            tw_hbm.at[pl.ds(0, _TWRND)], sbufs[buf], stage_sems[buf]
        ).wait()

    def sout(r, buf):
        pltpu.sync_copy(
            sbufs[buf], tw_sp.at[pl.ds(sid * _TWSEG + r * _TWRND, _TWRND)]
        )

    sin(0, 0)
    sin(1, 1)

    def stage_body(r, carry):
        buf = lax.rem(r, 2)
        for b in range(2):
            @pl.when(buf == b)
            def _():
                sin_wait(b)
                sout(r, b)

                @pl.when(r + 2 < nst)
                def _():
                    sin(r + 2, b)

        return carry

    lax.fori_loop(0, nst, stage_body, 0)

    @pl.when(sid == _NS - 1)
    def _():
        n_tail = VOCAB - _NS * _TWSEG
        pltpu.sync_copy(
            tw_hbm.at[pl.ds(_NS * _TWSEG, n_tail)],
            stage_v.at[pl.ds(0, n_tail)],
        )
        pltpu.sync_copy(
            stage_v.at[pl.ds(0, n_tail)],
            tw_sp.at[pl.ds(_NS * _TWSEG, n_tail)],
        )

    for p in range(2):
        bvec = bias_v[p, :]
        for i in range(_BPW // _L):
            accs[p][pl.ds(i * _L, _L)] = bvec

    def gather(c, slot):
        ring = lax.rem(c, _NRING) * _GE
        pltpu.async_copy(
            tw_sp.at[idx_v.at[pl.ds(ring, _GE)]], rows[slot], sems[slot]
        )

    def wait(slot):
        pltpu.make_async_copy(
            tw_sp.at[idx_v.at[pl.ds(0, _GE)]], rows[slot], sems[slot]
        ).wait()

    def accum(slot):
        for i in range(_BPW // _L):
            vs = [rows[slot][pl.ds(g * _BPW + i * _L, _L)]
                  for g in range(_GC)]
            his = [plsc.bitcast(lax.bitwise_and(v, -65536), jnp.float32)
                   for v in vs]
            los = [plsc.bitcast(lax.shift_left(v, 16), jnp.float32)
                   for v in vs]
            hi = (his[0] + his[1]) + (his[2] + his[3])
            lo = (los[0] + los[1]) + (los[2] + los[3])
            plsc.addupdate(accs[0].at[pl.ds(i * _L, _L)], hi)
            plsc.addupdate(accs[1].at[pl.ds(i * _L, _L)], lo)

    plsc.subcore_barrier()

    fill_wait(0)
    gather(0, 0)
    fill(4, 0)
    fill_wait(1)
    gather(1, 1)
    fill(5, 1)

    def body(i, carry):
        c0 = i * 2
        for slot in range(2):
            c = c0 + slot
            wait(slot)
            accum(slot)

            @pl.when(c + 2 < _NCHUNK)
            def _():
                fill_wait(slot)
                gather(c + 2, slot)

            @pl.when(c + 6 < _NCHUNK)
            def _():
                fill(c + 6, slot)

        return carry

    lax.fori_loop(0, _NCHUNK // 2, body, 0)

    pltpu.sync_copy(accs[0], out0_hbm.at[pl.ds(wid * _BPW, _BPW)])
    pltpu.sync_copy(accs[1], out1_hbm.at[pl.ds(wid * _BPW, _BPW)])


@functools.partial(
    pl.kernel,
    out_type=[
        jax.ShapeDtypeStruct((BATCH,), jnp.float32),
        jax.ShapeDtypeStruct((BATCH,), jnp.float32),
    ],
    mesh=plsc.VectorSubcoreMesh(core_axis_name="c", subcore_axis_name="s"),
    compiler_params=pltpu.CompilerParams(needs_layout_passes=False),
    scratch_types=[
        pltpu.VMEM((_NRING * _GE,), jnp.int32),
        pltpu.VMEM((_GE,), jnp.int32),
        pltpu.VMEM((_GE,), jnp.int32),
        pltpu.VMEM((_BPW,), jnp.float32),
        pltpu.VMEM((_BPW,), jnp.float32),
        pltpu.VMEM((2, _L), jnp.float32),
        pltpu.VMEM_SHARED((VOCAB,), jnp.int32),
        pltpu.VMEM((2 * _TWRND,), jnp.int32),
        pltpu.SemaphoreType.DMA,
        pltpu.SemaphoreType.DMA,
        pltpu.SemaphoreType.DMA,
        pltpu.SemaphoreType.DMA,
        pltpu.SemaphoreType.DMA,
        pltpu.SemaphoreType.DMA,
    ],
)
def _sc_gather(tw, xb3, bias, out0, out1, idx_v, r0, r1, acc0, acc1,
               bias_v, tw_sp, stage_v, fsem0, fsem1, sem0, sem1, ssem0,
               ssem1):
    _sc_body(tw, xb3, bias, out0, out1, idx_v, (r0, r1), (acc0, acc1),
             bias_v, tw_sp, stage_v, (fsem0, fsem1), (sem0, sem1),
             (ssem0, ssem1))


@jax.jit
def _run(xb, table, W, b):
    w_scaled = W * (1.0 / SEQ_LEN)
    tw = _compute_tw(w_scaled, table.T)
    xb3 = xb.reshape(SEQ_LEN, _NW, _BPW)
    bias2 = jnp.broadcast_to(b[:, None], (OUT_SIZE, _L))
    out0, out1 = _sc_gather(tw, xb3, bias2)
    return jnp.stack([out0, out1], axis=-1)


def kernel(xb, table, W, b):
    return _run(xb, table, W, b)


# speedup vs baseline: 1.0429x; 1.0016x over previous
import functools

import jax
import jax.numpy as jnp
from jax import lax
from jax.experimental import pallas as pl
from jax.experimental.pallas import tpu as pltpu
from jax.experimental.pallas import tpu_sc as plsc

VOCAB = 1000000
IN_SIZE = 64
OUT_SIZE = 2
SEQ_LEN = 200
BATCH = 16384


_VBLK = 32768


def _round_bf16_bits(x):
    bits = lax.bitcast_convert_type(x, jnp.int32)
    return bits + 0x7FFF + lax.bitwise_and(
        lax.shift_right_logical(bits, 16), 1
    )


def _tw_body(w_ref, tblt_ref, out_ref):
    res = lax.dot_general(
        w_ref[...],
        tblt_ref[...],
        dimension_numbers=(((1,), (0,)), ((), ())),
        preferred_element_type=jnp.float32,
    )
    hi = lax.bitwise_and(_round_bf16_bits(res[0, :]), -65536)
    lo = lax.shift_right_logical(_round_bf16_bits(res[1, :]), 16)
    out_ref[...] = lax.bitwise_or(hi, lo)


def _compute_tw(w_scaled, table_t):
    return pl.pallas_call(
        _tw_body,
        grid=(pl.cdiv(VOCAB, _VBLK),),
        in_specs=[
            pl.BlockSpec((OUT_SIZE, IN_SIZE), lambda i: (0, 0)),
            pl.BlockSpec((IN_SIZE, _VBLK), lambda i: (0, i)),
        ],
        out_specs=pl.BlockSpec((_VBLK,), lambda i: (i,)),
        out_shape=jax.ShapeDtypeStruct((VOCAB,), jnp.int32),
    )(w_scaled, table_t)


_NC = 2
_NS = 16
_NW = _NC * _NS
_BPW = BATCH // _NW
_L = 16
_GC = 4
_GE = _GC * _BPW
_NCHUNK = SEQ_LEN // _GC
_NRING = 8


_TWSEG = 62496
_TWRND = 8928


def _sc_body(tw_hbm, xb_hbm, bias_hbm, out0_hbm, out1_hbm,
             idx_v, rows, accs, bias_v, tw_sp, stage_v, fill_sems, sems,
             stage_sems):
    sid = lax.axis_index("s")
    wid = sid * _NC + lax.axis_index("c")


    def fill(c, slot):
        ring = lax.rem(c, _NRING) * _GE
        for j in range(_GC):
            l = c * _GC + j
            pltpu.async_copy(
                xb_hbm.at[l, wid, :],
                idx_v.at[pl.ds(ring + j * _BPW, _BPW)],
                fill_sems[slot],
            )

    def fill_wait(slot):
        for j in range(_GC):
            pltpu.make_async_copy(
                xb_hbm.at[0, 0, :], idx_v.at[pl.ds(0, _BPW)], fill_sems[slot]
            ).wait()

    fill(0, 0)
    fill(1, 1)

    nst = _TWSEG // _TWRND
    sbufs = (stage_v.at[pl.ds(0, _TWRND)], stage_v.at[pl.ds(_TWRND, _TWRND)])

    def sin(r, buf):
        pltpu.async_copy(
            tw_hbm.at[pl.ds(sid * _TWSEG + r * _TWRND, _TWRND)],
            sbufs[buf],
            stage_sems[buf],
        )

    def sin_wait(buf):
        pltpu.make_async_copy(
            tw_hbm.at[pl.ds(0, _TWRND)], sbufs[buf], stage_sems[buf]
        ).wait()

    def sout(r, buf):
        pltpu.sync_copy(
            sbufs[buf], tw_sp.at[pl.ds(sid * _TWSEG + r * _TWRND, _TWRND)]
        )

    sin(0, 0)
    sin(1, 1)

    def stage_body(r, carry):
        buf = lax.rem(r, 2)
        for b in range(2):
            @pl.when(buf == b)
            def _():
                sin_wait(b)
                sout(r, b)

                @pl.when(r + 2 < nst)
                def _():
                    sin(r + 2, b)

        return carry

    lax.fori_loop(0, nst, stage_body, 0)

    @pl.when(sid == _NS - 1)
    def _():
        n_tail = VOCAB - _NS * _TWSEG
        pltpu.sync_copy(
            tw_hbm.at[pl.ds(_NS * _TWSEG, n_tail)],
            stage_v.at[pl.ds(0, n_tail)],
        )
        pltpu.sync_copy(
            stage_v.at[pl.ds(0, n_tail)],
            tw_sp.at[pl.ds(_NS * _TWSEG, n_tail)],
        )

    for p in range(2):
        bvec = bias_v[p, :]
        for i in range(_BPW // _L):
            accs[p][pl.ds(i * _L, _L)] = bvec

    def gather(c, slot):
        ring = lax.rem(c, _NRING) * _GE
        pltpu.async_copy(
            tw_sp.at[idx_v.at[pl.ds(ring, _GE)]], rows[slot], sems[slot]
        )

    def wait(slot):
        pltpu.make_async_copy(
            tw_sp.at[idx_v.at[pl.ds(0, _GE)]], rows[slot], sems[slot]
        ).wait()

    def accum(slot):
        for i in range(_BPW // _L):
            vs = [rows[slot][pl.ds(g * _BPW + i * _L, _L)]
                  for g in range(_GC)]
            his = [plsc.bitcast(lax.bitwise_and(v, -65536), jnp.float32)
                   for v in vs]
            los = [plsc.bitcast(lax.shift_left(v, 16), jnp.float32)
                   for v in vs]
            hi = (his[0] + his[1]) + (his[2] + his[3])
            lo = (los[0] + los[1]) + (los[2] + los[3])
            plsc.addupdate(accs[0].at[pl.ds(i * _L, _L)], hi)
            plsc.addupdate(accs[1].at[pl.ds(i * _L, _L)], lo)

    plsc.subcore_barrier()

    fill_wait(0)
    gather(0, 0)
    fill(2, 0)
    fill_wait(1)
    gather(1, 1)
    fill(3, 1)

    def body(i, carry):
        c0 = i * 2
        for slot in range(2):
            c = c0 + slot
            wait(slot)
            accum(slot)

            @pl.when(c + 2 < _NCHUNK)
            def _():
                fill_wait(slot)
                gather(c + 2, slot)

            @pl.when(c + 4 < _NCHUNK)
            def _():
                fill(c + 4, slot)

        return carry

    lax.fori_loop(0, _NCHUNK // 2, body, 0)

    pltpu.sync_copy(accs[0], out0_hbm.at[pl.ds(wid * _BPW, _BPW)])
    pltpu.sync_copy(accs[1], out1_hbm.at[pl.ds(wid * _BPW, _BPW)])


@functools.partial(
    pl.kernel,
    out_type=[
        jax.ShapeDtypeStruct((BATCH,), jnp.float32),
        jax.ShapeDtypeStruct((BATCH,), jnp.float32),
    ],
    mesh=plsc.VectorSubcoreMesh(core_axis_name="c", subcore_axis_name="s"),
    compiler_params=pltpu.CompilerParams(needs_layout_passes=False),
    scratch_types=[
        pltpu.VMEM((_NRING * _GE,), jnp.int32),
        pltpu.VMEM((_GE,), jnp.int32),
        pltpu.VMEM((_GE,), jnp.int32),
        pltpu.VMEM((_BPW,), jnp.float32),
        pltpu.VMEM((_BPW,), jnp.float32),
        pltpu.VMEM((2, _L), jnp.float32),
        pltpu.VMEM_SHARED((VOCAB,), jnp.int32),
        pltpu.VMEM((2 * _TWRND,), jnp.int32),
        pltpu.SemaphoreType.DMA,
        pltpu.SemaphoreType.DMA,
        pltpu.SemaphoreType.DMA,
        pltpu.SemaphoreType.DMA,
        pltpu.SemaphoreType.DMA,
        pltpu.SemaphoreType.DMA,
    ],
)
def _sc_gather(tw, xb3, bias, out0, out1, idx_v, r0, r1, acc0, acc1,
               bias_v, tw_sp, stage_v, fsem0, fsem1, sem0, sem1, ssem0,
               ssem1):
    _sc_body(tw, xb3, bias, out0, out1, idx_v, (r0, r1), (acc0, acc1),
             bias_v, tw_sp, stage_v, (fsem0, fsem1), (sem0, sem1),
             (ssem0, ssem1))


@jax.jit
def _run(xb, table, W, b):
    w_scaled = W * (1.0 / SEQ_LEN)
    tw = _compute_tw(w_scaled, table.T)
    xb3 = xb.reshape(SEQ_LEN, _NW, _BPW)
    bias2 = jnp.broadcast_to(b[:, None], (OUT_SIZE, _L))
    out0, out1 = _sc_gather(tw, xb3, bias2)
    return jnp.stack([out0, out1], axis=-1)


def kernel(xb, table, W, b):
    return _run(xb, table, W, b)
